# baseline (speedup 1.0000x reference)
import math

import jax
import jax.numpy as jnp
import numpy as np
from jax.experimental import pallas as pl

_S, _N, _NP = 8, 97, 104
_R = _S * _NP


def _pe8_np():
    pos = np.arange(20, dtype=np.float32)[:, None]
    div = np.exp(np.arange(0, 16, 2, dtype=np.float32) * (-math.log(10000.0) / 16.0))
    pe = np.zeros((20, 16), dtype=np.float32)
    pe[:, 0::2] = np.sin(pos * div)
    pe[:, 1::2] = np.cos(pos * div)
    return pe[:_S]


_PE8 = _pe8_np()

_W_SPECS = [
    ("emb1", (8, 3)), ("emb2", (5, 3)), ("lin0_w", (4, 3)), ("lin1_w", (4, 3)),
    ("lin2_w", (8, 1)), ("gc10_w", (16, 64)), ("gc11_w", (64, 32)),
    ("gc20_w", (16, 64)), ("gc21_w", (64, 32)), ("gc30_w", (16, 64)),
    ("gc31_w", (64, 32)), ("fw0", (32, 16)), ("fw1", (32, 16)), ("fw2", (32, 16)),
    ("attn_in_w", (48, 16)), ("attn_out_w", (16, 16)), ("pred_w", (8, 16)),
    ("out0_w", (4, 8)), ("out1_w", (1, 4)), ("pe", (8, 16)),
]
_W_OFF = {}
_off = 0
for _name, (_r, _c) in _W_SPECS:
    _W_OFF[_name] = (_off, _r, _c)
    _off += _r + ((-_r) % 8)
_W_ROWS = _off

_B_SPECS = [
    ("lin0_b", 4), ("lin1_b", 4), ("lin2_b", 8), ("gc10_b", 64), ("gc11_b", 32),
    ("gc20_b", 64), ("gc21_b", 32), ("gc30_b", 64), ("gc31_b", 32),
    ("gcn_ln_g", 16), ("gcn_ln_b", 16), ("attn_in_b", 48), ("attn_out_b", 16),
    ("norm1_g", 16), ("norm1_b", 16), ("norm2_g", 16), ("norm2_b", 16),
    ("enc_norm_g", 16), ("enc_norm_b", 16), ("pred_b", 8), ("out0_b", 4),
    ("out1_b", 1), ("ff2_b", 16),
]
_B_OFF = {n: (128 * i, w) for i, (n, w) in enumerate(_B_SPECS)}
_B_LANES = 128 * len(_B_SPECS)


def _mm_t(x, w):
    return jax.lax.dot_general(
        x, w, (((1,), (1,)), ((), ())), preferred_element_type=jnp.float32
    )


def _mm(x, w):
    return jax.lax.dot_general(
        x, w, (((1,), (0,)), ((), ())), preferred_element_type=jnp.float32
    )


def _ln(x, g, b, eps):
    m = jnp.mean(x, axis=-1, keepdims=True)
    v = jnp.mean((x - m) * (x - m), axis=-1, keepdims=True)
    return (x - m) * jax.lax.rsqrt(v + eps) * g + b


def _sel(rows, cols, shift):
    r = jax.lax.broadcasted_iota(jnp.int32, (rows, cols), 0)
    c = jax.lax.broadcasted_iota(jnp.int32, (rows, cols), 1)
    return (c == r + shift).astype(jnp.float32)


def _blk(x, i):
    return x[i * _NP : (i + 1) * _NP, :]


def _fused_body(feat, ws, apack, wpack, bpack, ff1_w, ff1_b, ff2_w, r1_ref, r2_ref):
    f32 = jnp.float32

    def W(name):
        off, r, c = _W_OFF[name]
        return wpack[off : off + r, 0:c]

    def B(name):
        off, w = _B_OFF[name]
        return bpack[:, off : off + w]

    A0, A1, A2 = apack[0], apack[1], apack[2]
    week, stamp = ws[0], ws[1]

    w3f = _mm(jnp.transpose(W("lin2_w")), _sel(8, 16, 0))
    w2f = _mm(_mm_t(W("emb2"), W("lin1_w")), _sel(4, 16, 8))
    w1f = _mm(_mm_t(W("emb1"), W("lin0_w")), _sel(4, 16, 12))
    bias16 = (
        _mm(B("lin2_b"), _sel(8, 16, 0))
        + _mm(B("lin1_b"), _sel(4, 16, 8))
        + _mm(B("lin0_b"), _sel(4, 16, 12))
    )

    oh_w = (week == jax.lax.broadcasted_iota(jnp.int32, (_R, 8), 1)).astype(f32)
    oh_s = (stamp == jax.lax.broadcasted_iota(jnp.int32, (_R, 5), 1)).astype(f32)
    X = _mm(feat[:, :], w3f) + _mm(oh_s, w2f) + _mm(oh_w, w1f) + bias16

    def gcn_branch(A, w0, b0, w1, b1):
        U = _mm(X, w0)
        V = jnp.concatenate([_mm(A, _blk(U, i)) for i in range(_S)], axis=0)
        H = jnp.maximum(V + b0, 0.0)
        Wd = _mm(H, w1)
        Z = jnp.concatenate([_mm(A, _blk(Wd, i)) for i in range(_S)], axis=0)
        return Z + b1

    z0 = gcn_branch(A0, W("gc10_w"), B("gc10_b"), W("gc11_w"), B("gc11_b"))
    z1 = gcn_branch(A1, W("gc20_w"), B("gc20_b"), W("gc21_w"), B("gc21_b"))
    z2 = gcn_branch(A2, W("gc30_w"), B("gc30_b"), W("gc31_w"), B("gc31_b"))
    xo = _mm(z0, W("fw0")) + _mm(z1, W("fw1")) + _mm(z2, W("fw2"))
    xg = _ln(xo + X, B("gcn_ln_g"), B("gcn_ln_b"), 1e-6)

    step_oh = (
        jax.lax.broadcasted_iota(jnp.int32, (_R, 8), 0) // _NP
        == jax.lax.broadcasted_iota(jnp.int32, (_R, 8), 1)
    ).astype(f32)
    src = xg + _mm(step_oh, W("pe"))

    attn_in_w = W("attn_in_w")
    wq = attn_in_w[0:16, :]
    wk = attn_in_w[16:32, :]
    wv = attn_in_w[32:48, :]
    attn_in_b = B("attn_in_b")
    bq = _mm(attn_in_b, _sel(16, 48, 0).T)
    bk = _mm(attn_in_b, _sel(16, 48, 16).T)
    bv = _mm(attn_in_b, _sel(16, 48, 32).T)
    q_all = _mm_t(src, wq) + bq
    k_all = _mm_t(src, wk) + bk
    v_all = _mm_t(src, wv) + bv

    G = (
        jax.lax.broadcasted_iota(jnp.int32, (16, 4), 0) // 4
        == jax.lax.broadcasted_iota(jnp.int32, (16, 4), 1)
    ).astype(f32)

    def tile_steps(x):
        return jnp.concatenate([x] * _S, axis=0)

    scores = []
    for j in range(_S):
        kt = tile_steps(_blk(k_all, j))
        scores.append(_mm(q_all * kt, G) * 0.5)
    m = scores[0]
    for j in range(1, _S):
        m = jnp.maximum(m, scores[j])
    exps = [jnp.exp(s - m) for s in scores]
    den = exps[0]
    for j in range(1, _S):
        den = den + exps[j]
    inv = 1.0 / den
    ao = jnp.zeros((_R, 16), f32)
    for j in range(_S):
        vt = tile_steps(_blk(v_all, j))
        ao = ao + _mm_t(exps[j] * inv, G) * vt

    ao = _mm_t(ao, W("attn_out_w")) + B("attn_out_b")
    x1 = _ln(src + ao, B("norm1_g"), B("norm1_b"), 1e-5)
    h = jnp.maximum(_mm_t(x1, ff1_w[:, :]) + ff1_b[:, :], 0.0)
    y = _mm_t(h, ff2_w[:, :]) + B("ff2_b")
    x2 = _ln(x1 + y, B("norm2_g"), B("norm2_b"), 1e-5)
    enc = _ln(x2, B("enc_norm_g"), B("enc_norm_b"), 1e-6)

    r1 = _mm_t(enc, W("pred_w")) + B("pred_b")
    rb = _mm_t(r1, W("out0_w")) + B("out0_b")
    r2 = jnp.sum(rb * W("out1_w"), axis=-1, keepdims=True) + B("out1_b")[0, 0]
    r1_ref[:, :] = r1
    r2_ref[:, :] = r2


def kernel(feature_tensor, week_tensor, stamptensor, a0, a1, a2, k, params):
    p = params
    del k
    feat = jnp.pad(feature_tensor, ((0, 0), (0, _NP - _N))).reshape(_R, 1)
    ws = (
        jnp.zeros((2, _S, _NP), jnp.int32)
        .at[:, :, :_N]
        .set(jnp.stack([week_tensor, stamptensor]))
        .reshape(2, _R, 1)
    )
    apack = (
        jnp.zeros((3, _NP, _NP), jnp.float32)
        .at[:, :_N, :_N]
        .set(jnp.stack([a0, a1, a2]))
    )

    wpieces = []
    for name, (r, c) in _W_SPECS:
        a = jnp.asarray(_PE8) if name == "pe" else p[name]
        wpieces.append(jnp.pad(a, ((0, (-r) % 8), (0, 64 - c))))
    wpack = jnp.concatenate(wpieces, axis=0)

    bpieces = [
        jnp.pad(p[name].reshape(1, -1), ((0, 0), (0, 128 - w)))
        for name, w in _B_SPECS
    ]
    bpack = jnp.concatenate(bpieces, axis=1)

    r1p, r2p = pl.pallas_call(
        _fused_body,
        out_shape=[
            jax.ShapeDtypeStruct((_R, 8), jnp.float32),
            jax.ShapeDtypeStruct((_R, 1), jnp.float32),
        ],
    )(feat, ws, apack, wpack, bpack, p["ff1_w"], p["ff1_b"].reshape(1, 2048), p["ff2_w"])
    r1 = r1p.reshape(_S, _NP, 8)[:, :_N, :]
    r2 = r2p.reshape(_S, _NP, 1)[:, :_N, :]
    return (r1, r2, r2[-1])


# speedup vs baseline: 1.7771x; 1.7771x over previous
import math

import jax
import jax.numpy as jnp
import numpy as np
from jax.experimental import pallas as pl

_S, _N, _NP = 8, 97, 104
_R = _S * _NP


def _pe8_np():
    pos = np.arange(20, dtype=np.float32)[:, None]
    div = np.exp(np.arange(0, 16, 2, dtype=np.float32) * (-math.log(10000.0) / 16.0))
    pe = np.zeros((20, 16), dtype=np.float32)
    pe[:, 0::2] = np.sin(pos * div)
    pe[:, 1::2] = np.cos(pos * div)
    return pe[:_S]


_PE8 = _pe8_np()


def _mm_t(x, w):
    return jax.lax.dot_general(
        x, w, (((1,), (1,)), ((), ())), preferred_element_type=jnp.float32
    )


def _mm(x, w):
    return jax.lax.dot_general(
        x, w, (((1,), (0,)), ((), ())), preferred_element_type=jnp.float32
    )


def _ln(x, g, b, eps):
    m = jnp.mean(x, axis=-1, keepdims=True)
    v = jnp.mean((x - m) * (x - m), axis=-1, keepdims=True)
    return (x - m) * jax.lax.rsqrt(v + eps) * g + b


def _sel(rows, cols, shift):
    r = jax.lax.broadcasted_iota(jnp.int32, (rows, cols), 0)
    c = jax.lax.broadcasted_iota(jnp.int32, (rows, cols), 1)
    return (c == r + shift).astype(jnp.float32)


def _blk(x, i):
    return x[i * _NP : (i + 1) * _NP, :]


def _fused_body(
    feat, week, stamp, a0, a1, a2, pe,
    emb1, emb2, lin0_w, lin1_w, lin2_w, lin0_b, lin1_b, lin2_b,
    gc10_w, gc10_b, gc11_w, gc11_b,
    gc20_w, gc20_b, gc21_w, gc21_b,
    gc30_w, gc30_b, gc31_w, gc31_b,
    fw0, fw1, fw2, gcn_g, gcn_b,
    attn_in_w, attn_in_b, attn_out_w, attn_out_b,
    n1_g, n1_b, ff1_w, ff1_b, ff2_w, ff2_b, n2_g, n2_b, en_g, en_b,
    pred_w, pred_b, out0_w, out0_b, out1_w, out1_b,
    r1_ref, r2_ref, r2l_ref,
):
    f32 = jnp.float32
    A0 = jnp.pad(a0[:, :], ((0, _NP - _N), (0, _NP - _N)))
    A1 = jnp.pad(a1[:, :], ((0, _NP - _N), (0, _NP - _N)))
    A2 = jnp.pad(a2[:, :], ((0, _NP - _N), (0, _NP - _N)))

    featc = jnp.pad(jnp.transpose(feat[:, :]), ((0, _NP - _N), (0, 0)))
    weekc = jnp.pad(jnp.transpose(week[:, :]), ((0, _NP - _N), (0, 0)))
    stampc = jnp.pad(jnp.transpose(stamp[:, :]), ((0, _NP - _N), (0, 0)))
    fcol = jnp.concatenate([featc[:, i : i + 1] for i in range(_S)], axis=0)
    oh_w = jnp.concatenate(
        [
            (weekc[:, i : i + 1] == jax.lax.broadcasted_iota(jnp.int32, (_NP, 8), 1)).astype(f32)
            for i in range(_S)
        ],
        axis=0,
    )
    oh_s = jnp.concatenate(
        [
            (stampc[:, i : i + 1] == jax.lax.broadcasted_iota(jnp.int32, (_NP, 5), 1)).astype(f32)
            for i in range(_S)
        ],
        axis=0,
    )

    w3f = _mm(jnp.transpose(lin2_w[:, :]), _sel(8, 16, 0))
    w2f = _mm(_mm_t(emb2[:, :], lin1_w[:, :]), _sel(4, 16, 8))
    w1f = _mm(_mm_t(emb1[:, :], lin0_w[:, :]), _sel(4, 16, 12))
    bias16 = (
        _mm(lin2_b[:, :], _sel(8, 16, 0))
        + _mm(lin1_b[:, :], _sel(4, 16, 8))
        + _mm(lin0_b[:, :], _sel(4, 16, 12))
    )
    X = _mm(fcol, w3f) + _mm(oh_s, w2f) + _mm(oh_w, w1f) + bias16

    def gcn_branch(A, w0, b0, w1, b1):
        U = _mm(X, w0[:, :])
        V = jnp.concatenate([_mm(A, _blk(U, i)) for i in range(_S)], axis=0)
        H = jnp.maximum(V + b0[:, :], 0.0)
        Wd = _mm(H, w1[:, :])
        Z = jnp.concatenate([_mm(A, _blk(Wd, i)) for i in range(_S)], axis=0)
        return Z + b1[:, :]

    z0 = gcn_branch(A0, gc10_w, gc10_b, gc11_w, gc11_b)
    z1 = gcn_branch(A1, gc20_w, gc20_b, gc21_w, gc21_b)
    z2 = gcn_branch(A2, gc30_w, gc30_b, gc31_w, gc31_b)
    xo = _mm(z0, fw0[:, :]) + _mm(z1, fw1[:, :]) + _mm(z2, fw2[:, :])
    xg = _ln(xo + X, gcn_g[:, :], gcn_b[:, :], 1e-6)

    step_oh = (
        jax.lax.broadcasted_iota(jnp.int32, (_R, 8), 0) // _NP
        == jax.lax.broadcasted_iota(jnp.int32, (_R, 8), 1)
    ).astype(f32)
    src = xg + _mm(step_oh, pe[:, :])

    wq = attn_in_w[0:16, :]
    wk = attn_in_w[16:32, :]
    wv = attn_in_w[32:48, :]
    bq = _mm(attn_in_b[:, :], _sel(16, 48, 0).T)
    bk = _mm(attn_in_b[:, :], _sel(16, 48, 16).T)
    bv = _mm(attn_in_b[:, :], _sel(16, 48, 32).T)
    q_all = _mm_t(src, wq) + bq
    k_all = _mm_t(src, wk) + bk
    v_all = _mm_t(src, wv) + bv

    G = (
        jax.lax.broadcasted_iota(jnp.int32, (16, 4), 0) // 4
        == jax.lax.broadcasted_iota(jnp.int32, (16, 4), 1)
    ).astype(f32)

    def tile_steps(x):
        return jnp.concatenate([x] * _S, axis=0)

    scores = []
    for j in range(_S):
        kt = tile_steps(_blk(k_all, j))
        scores.append(_mm(q_all * kt, G) * 0.5)
    m = scores[0]
    for j in range(1, _S):
        m = jnp.maximum(m, scores[j])
    exps = [jnp.exp(s - m) for s in scores]
    den = exps[0]
    for j in range(1, _S):
        den = den + exps[j]
    inv = 1.0 / den
    ao = jnp.zeros((_R, 16), f32)
    for j in range(_S):
        vt = tile_steps(_blk(v_all, j))
        ao = ao + _mm_t(exps[j] * inv, G) * vt

    ao = _mm_t(ao, attn_out_w[:, :]) + attn_out_b[:, :]
    x1 = _ln(src + ao, n1_g[:, :], n1_b[:, :], 1e-5)
    h = jnp.maximum(_mm_t(x1, ff1_w[:, :]) + ff1_b[:, :], 0.0)
    y = _mm_t(h, ff2_w[:, :]) + ff2_b[:, :]
    x2 = _ln(x1 + y, n2_g[:, :], n2_b[:, :], 1e-5)
    enc = _ln(x2, en_g[:, :], en_b[:, :], 1e-6)

    r1 = _mm_t(enc, pred_w[:, :]) + pred_b[:, :]
    rb = _mm_t(r1, out0_w[:, :]) + out0_b[:, :]
    r2 = jnp.sum(rb * out1_w[:, :], axis=-1, keepdims=True) + out1_b[0, 0]
    for i in range(_S):
        r1_ref[i] = r1[i * _NP : i * _NP + _N, :]
        r2_ref[i] = r2[i * _NP : i * _NP + _N, :]
    r2l_ref[:, :] = r2[(_S - 1) * _NP : (_S - 1) * _NP + _N, :]


def kernel(feature_tensor, week_tensor, stamptensor, a0, a1, a2, k, params):
    p = params
    del k
    return tuple(
        pl.pallas_call(
            _fused_body,
            out_shape=[
                jax.ShapeDtypeStruct((_S, _N, 8), jnp.float32),
                jax.ShapeDtypeStruct((_S, _N, 1), jnp.float32),
                jax.ShapeDtypeStruct((_N, 1), jnp.float32),
            ],
        )(
            feature_tensor, week_tensor, stamptensor, a0, a1, a2, jnp.asarray(_PE8),
            p["emb1"], p["emb2"], p["lin0_w"], p["lin1_w"], p["lin2_w"],
            p["lin0_b"].reshape(1, 4), p["lin1_b"].reshape(1, 4), p["lin2_b"].reshape(1, 8),
            p["gc10_w"], p["gc10_b"].reshape(1, 64), p["gc11_w"], p["gc11_b"].reshape(1, 32),
            p["gc20_w"], p["gc20_b"].reshape(1, 64), p["gc21_w"], p["gc21_b"].reshape(1, 32),
            p["gc30_w"], p["gc30_b"].reshape(1, 64), p["gc31_w"], p["gc31_b"].reshape(1, 32),
            p["fw0"], p["fw1"], p["fw2"],
            p["gcn_ln_g"].reshape(1, 16), p["gcn_ln_b"].reshape(1, 16),
            p["attn_in_w"], p["attn_in_b"].reshape(1, 48),
            p["attn_out_w"], p["attn_out_b"].reshape(1, 16),
            p["norm1_g"].reshape(1, 16), p["norm1_b"].reshape(1, 16),
            p["ff1_w"], p["ff1_b"].reshape(1, 2048),
            p["ff2_w"], p["ff2_b"].reshape(1, 16),
            p["norm2_g"].reshape(1, 16), p["norm2_b"].reshape(1, 16),
            p["enc_norm_g"].reshape(1, 16), p["enc_norm_b"].reshape(1, 16),
            p["pred_w"], p["pred_b"].reshape(1, 8),
            p["out0_w"], p["out0_b"].reshape(1, 4),
            p["out1_w"], p["out1_b"].reshape(1, 1),
        )
    )
